# baseline (speedup 1.0000x reference)
import jax
import jax.numpy as jnp
from jax.experimental import pallas as pl

N = 35
HID = 120
H = 16
_NEG = -1e30


def _expand_attn(a):
    Hh, C = a.shape
    eye = jnp.eye(Hh, dtype=a.dtype)
    return (a[:, :, None] * eye[:, None, :]).reshape(Hh * C, Hh)


def _gat_kernel(adj_ref, W1_ref, As1_ref, Ad1_ref, b1_ref,
                W2a_ref, W2b_ref, W2c_ref, W2d_ref,
                As2_ref, Ad2_ref, b2_ref,
                W3_ref, as3_ref, ad3_ref, b3_ref,
                Wfc_ref, bfc_ref, out_ref):
    f32 = jnp.float32
    adj = adj_ref[:]
    ii = jax.lax.broadcasted_iota(jnp.int32, (N, N), 0)
    jj = jax.lax.broadcasted_iota(jnp.int32, (N, N), 1)
    countf = (adj > 0.5).astype(f32) + (ii == jj).astype(f32)
    has_edge = countf > 0.0

    def attn(h, a_s, a_dT, heads, C):
        outs = []
        for hd in range(heads):
            e = a_s[:, hd:hd + 1] + a_dT[hd:hd + 1, :]
            e = jnp.where(e >= 0.0, e, 0.2 * e)
            e = jnp.where(has_edge, e, _NEG)
            m = jnp.max(e, axis=0, keepdims=True)
            ex = jnp.exp(e - m) * countf
            s = jnp.sum(ex, axis=0, keepdims=True)
            p = ex / (s + 1e-16)
            hs = h[:, hd * C:(hd + 1) * C]
            outs.append(jax.lax.dot_general(
                p, hs, (((0,), (0,)), ((), ())), preferred_element_type=f32))
        return jnp.concatenate(outs, axis=1) if heads > 1 else outs[0]

    def layer(x, W, As, Ad, b, heads, C):
        if isinstance(W, (list, tuple)):
            h = jnp.concatenate(
                [jnp.dot(x, w, preferred_element_type=f32) for w in W], axis=1)
        else:
            h = jnp.dot(x, W, preferred_element_type=f32)
        a_s = jnp.dot(h, As, preferred_element_type=f32)
        a_dT = jax.lax.dot_general(
            Ad, h, (((0,), (1,)), ((), ())), preferred_element_type=f32)
        return attn(h, a_s, a_dT, heads, C) + b

    x = layer(adj, W1_ref[:], As1_ref[:], Ad1_ref[:], b1_ref[:], H, HID)
    x = jnp.where(x > 0.0, x, jnp.exp(jnp.minimum(x, 0.0)) - 1.0)
    x = layer(x, (W2a_ref[:], W2b_ref[:], W2c_ref[:], W2d_ref[:]),
              As2_ref[:], Ad2_ref[:], b2_ref[:], H, HID)
    x = jnp.where(x > 0.0, x, jnp.exp(jnp.minimum(x, 0.0)) - 1.0)
    x = layer(x, W3_ref[:], as3_ref[:], ad3_ref[:], b3_ref[:], 1, HID)
    out = jnp.dot(x, Wfc_ref[:], preferred_element_type=f32) + bfc_ref[:]
    out_ref[:] = jnp.maximum(out, 0.0)


def kernel(adj_matrix, W1, as1, ad1, b1, W2, as2, ad2, b2,
           W3, as3, ad3, b3, Wfc, bfc):
    As1 = _expand_attn(as1)
    Ad1 = _expand_attn(ad1)
    As2 = _expand_attn(as2)
    Ad2 = _expand_attn(ad2)
    as3T = as3.T
    ad3T = ad3.T
    return pl.pallas_call(
        _gat_kernel,
        out_shape=jax.ShapeDtypeStruct((N, N), jnp.float32),
    )(adj_matrix, W1, As1, Ad1, b1.reshape(1, -1),
      W2[:, :480], W2[:, 480:960], W2[:, 960:1440], W2[:, 1440:],
      As2, Ad2, b2.reshape(1, -1),
      W3, as3T, ad3T, b3.reshape(1, -1),
      Wfc, bfc.reshape(1, -1))


# speedup vs baseline: 2.3039x; 2.3039x over previous
import jax
import jax.numpy as jnp
from jax.experimental import pallas as pl

N = 35
HID = 120
H = 16
_NEG = -1e30


def _gat_kernel(adj_ref, W1_ref, as1_ref, ad1_ref, b1_ref,
                W2_ref, as2_ref, ad2_ref, b2_ref,
                W3_ref, as3_ref, ad3_ref, b3_ref,
                Wfc_ref, bfc_ref, out_ref):
    f32 = jnp.float32
    adj = adj_ref[:]
    ii = jax.lax.broadcasted_iota(jnp.int32, (N, N), 0)
    jj = jax.lax.broadcasted_iota(jnp.int32, (N, N), 1)
    countf = (adj > 0.5).astype(f32) + (ii == jj).astype(f32)
    has_edge = countf > 0.0

    def layer(x, W, a_src, a_dst, b, heads, C):
        h = jnp.dot(x, W, preferred_element_type=f32)
        outs = []
        for hd in range(heads):
            hs = h[:, hd * C:(hd + 1) * C]
            asr = a_src[hd:hd + 1, :]
            adr = a_dst[hd:hd + 1, :]
            col = jax.lax.dot_general(
                hs, asr, (((1,), (1,)), ((), ())), preferred_element_type=f32)
            row = jax.lax.dot_general(
                adr, hs, (((1,), (1,)), ((), ())), preferred_element_type=f32)
            e = col + row
            e = jnp.where(e >= 0.0, e, 0.2 * e)
            e = jnp.where(has_edge, e, _NEG)
            m = jnp.max(e, axis=0, keepdims=True)
            ex = jnp.exp(e - m) * countf
            s = jnp.sum(ex, axis=0, keepdims=True)
            p = ex / (s + 1e-16)
            outs.append(jax.lax.dot_general(
                p, hs, (((0,), (0,)), ((), ())), preferred_element_type=f32))
        out = jnp.concatenate(outs, axis=1) if heads > 1 else outs[0]
        return out + jnp.reshape(b, (1, heads * C))

    x = layer(adj, W1_ref[:], as1_ref[:], ad1_ref[:], b1_ref[:], H, HID)
    x = jnp.where(x > 0.0, x, jnp.exp(jnp.minimum(x, 0.0)) - 1.0)
    x = layer(x, W2_ref[:], as2_ref[:], ad2_ref[:], b2_ref[:], H, HID)
    x = jnp.where(x > 0.0, x, jnp.exp(jnp.minimum(x, 0.0)) - 1.0)
    x = layer(x, W3_ref[:], as3_ref[:], ad3_ref[:], b3_ref[:], 1, HID)
    out = (jnp.dot(x, Wfc_ref[:], preferred_element_type=f32)
           + jnp.reshape(bfc_ref[:], (1, N)))
    out_ref[:] = jnp.maximum(out, 0.0)


def kernel(adj_matrix, W1, as1, ad1, b1, W2, as2, ad2, b2,
           W3, as3, ad3, b3, Wfc, bfc):
    return pl.pallas_call(
        _gat_kernel,
        out_shape=jax.ShapeDtypeStruct((N, N), jnp.float32),
    )(adj_matrix, W1, as1, ad1, b1, W2, as2, ad2, b2,
      W3, as3, ad3, b3, Wfc, bfc)


# speedup vs baseline: 2.5126x; 1.0906x over previous
import jax
import jax.numpy as jnp
from jax.experimental import pallas as pl

N = 35
HID = 120
H = 16
_NEG = -1e30
_Q = 4
_QW = H * HID // _Q


def _gat_kernel(adj_ref, W1_ref, as1_ref, ad1_ref, b1_ref,
                W2a_ref, W2b_ref, W2c_ref, W2d_ref,
                as2_ref, ad2_ref, b2_ref,
                W3_ref, as3_ref, ad3_ref, b3_ref,
                Wfc_ref, bfc_ref, out_ref):
    f32 = jnp.float32
    adj = adj_ref[:]
    ii = jax.lax.broadcasted_iota(jnp.int32, (N, N), 0)
    jj = jax.lax.broadcasted_iota(jnp.int32, (N, N), 1)
    countf = (adj > 0.5).astype(f32) + (ii == jj).astype(f32)
    has_edge = countf > 0.0

    def heads_block(h, a_src, a_dst, head_ids, C):
        outs = []
        for k, hd in enumerate(head_ids):
            hs = h[:, k * C:(k + 1) * C]
            asr = a_src[hd:hd + 1, :]
            adr = a_dst[hd:hd + 1, :]
            col = jax.lax.dot_general(
                hs, asr, (((1,), (1,)), ((), ())), preferred_element_type=f32)
            row = jax.lax.dot_general(
                adr, hs, (((1,), (1,)), ((), ())), preferred_element_type=f32)
            e = col + row
            e = jnp.where(e >= 0.0, e, 0.2 * e)
            e = jnp.where(has_edge, e, _NEG)
            m = jnp.max(e, axis=0, keepdims=True)
            ex = jnp.exp(e - m) * countf
            s = jnp.sum(ex, axis=0, keepdims=True)
            p = ex / (s + 1e-16)
            outs.append(jax.lax.dot_general(
                p, hs, (((0,), (0,)), ((), ())), preferred_element_type=f32))
        return outs

    def elu(x):
        return jnp.where(x > 0.0, x, jnp.exp(jnp.minimum(x, 0.0)) - 1.0)

    h1 = jnp.dot(adj, W1_ref[:], preferred_element_type=f32)
    o1 = heads_block(h1, as1_ref[:], ad1_ref[:], list(range(H)), HID)
    x1 = elu(jnp.concatenate(o1, axis=1) + jnp.reshape(b1_ref[:], (1, H * HID)))

    h2 = None
    for q, wref in enumerate((W2a_ref, W2b_ref, W2c_ref, W2d_ref)):
        part = jnp.dot(x1[:, q * _QW:(q + 1) * _QW], wref[:],
                       preferred_element_type=f32)
        h2 = part if h2 is None else h2 + part
    o2 = heads_block(h2, as2_ref[:], ad2_ref[:], list(range(H)), HID)
    x2 = elu(jnp.concatenate(o2, axis=1) + jnp.reshape(b2_ref[:], (1, H * HID)))

    h3 = jnp.dot(x2, W3_ref[:], preferred_element_type=f32)
    o3 = heads_block(h3, as3_ref[:], ad3_ref[:], [0], HID)[0]
    x3 = o3 + jnp.reshape(b3_ref[:], (1, HID))
    out = (jnp.dot(x3, Wfc_ref[:], preferred_element_type=f32)
           + jnp.reshape(bfc_ref[:], (1, N)))
    out_ref[:] = jnp.maximum(out, 0.0)


def _full(shape):
    nd = len(shape)
    return pl.BlockSpec(shape, lambda i: (0,) * nd)


def kernel(adj_matrix, W1, as1, ad1, b1, W2, as2, ad2, b2,
           W3, as3, ad3, b3, Wfc, bfc):
    KC = H * HID
    w2_specs = [pl.BlockSpec((KC // _Q, KC), lambda i, q=q: (q, 0))
                for q in range(_Q)]
    in_specs = [
        _full((N, N)), _full((N, KC)), _full((H, HID)), _full((H, HID)),
        _full((KC,)),
        *w2_specs,
        _full((H, HID)), _full((H, HID)), _full((KC,)),
        _full((KC, HID)), _full((1, HID)), _full((1, HID)), _full((HID,)),
        _full((HID, N)), _full((N,)),
    ]
    return pl.pallas_call(
        _gat_kernel,
        out_shape=jax.ShapeDtypeStruct((N, N), jnp.float32),
        grid=(1,),
        in_specs=in_specs,
        out_specs=_full((N, N)),
    )(adj_matrix, W1, as1, ad1, b1, W2, W2, W2, W2, as2, ad2, b2,
      W3, as3, ad3, b3, Wfc, bfc)


# speedup vs baseline: 2.6180x; 1.0420x over previous
import jax
import jax.numpy as jnp
from jax.experimental import pallas as pl

N = 35
HID = 120
H = 16
_NEG = -1e30
_Q = 8
_QW = H * HID // _Q


def _gat_kernel(adj_ref, W1_ref, as1_ref, ad1_ref, b1_ref,
                *rest):
    w2_refs = rest[:_Q]
    (as2_ref, ad2_ref, b2_ref, W3_ref, as3_ref, ad3_ref, b3_ref,
     Wfc_ref, bfc_ref, out_ref) = rest[_Q:]
    f32 = jnp.float32
    adj = adj_ref[:]
    ii = jax.lax.broadcasted_iota(jnp.int32, (N, N), 0)
    jj = jax.lax.broadcasted_iota(jnp.int32, (N, N), 1)
    countf = (adj > 0.5).astype(f32) + (ii == jj).astype(f32)
    has_edge = countf > 0.0

    def heads_block(h, a_src, a_dst, head_ids, C):
        outs = []
        for k, hd in enumerate(head_ids):
            hs = h[:, k * C:(k + 1) * C]
            asr = a_src[hd:hd + 1, :]
            adr = a_dst[hd:hd + 1, :]
            col = jax.lax.dot_general(
                hs, asr, (((1,), (1,)), ((), ())), preferred_element_type=f32)
            row = jax.lax.dot_general(
                adr, hs, (((1,), (1,)), ((), ())), preferred_element_type=f32)
            e = col + row
            e = jnp.where(e >= 0.0, e, 0.2 * e)
            e = jnp.where(has_edge, e, _NEG)
            m = jnp.max(e, axis=0, keepdims=True)
            ex = jnp.exp(e - m) * countf
            s = jnp.sum(ex, axis=0, keepdims=True)
            p = ex / (s + 1e-16)
            outs.append(jax.lax.dot_general(
                p, hs, (((0,), (0,)), ((), ())), preferred_element_type=f32))
        return outs

    def elu(x):
        return jnp.where(x > 0.0, x, jnp.exp(jnp.minimum(x, 0.0)) - 1.0)

    h1 = jnp.dot(adj, W1_ref[:], preferred_element_type=f32)
    o1 = heads_block(h1, as1_ref[:], ad1_ref[:], list(range(H)), HID)
    x1 = elu(jnp.concatenate(o1, axis=1) + jnp.reshape(b1_ref[:], (1, H * HID)))

    h2 = None
    for q, wref in enumerate(w2_refs):
        part = jnp.dot(x1[:, q * _QW:(q + 1) * _QW], wref[:],
                       preferred_element_type=f32)
        h2 = part if h2 is None else h2 + part
    o2 = heads_block(h2, as2_ref[:], ad2_ref[:], list(range(H)), HID)
    x2 = elu(jnp.concatenate(o2, axis=1) + jnp.reshape(b2_ref[:], (1, H * HID)))

    h3 = jnp.dot(x2, W3_ref[:], preferred_element_type=f32)
    o3 = heads_block(h3, as3_ref[:], ad3_ref[:], [0], HID)[0]
    x3 = o3 + jnp.reshape(b3_ref[:], (1, HID))
    out = (jnp.dot(x3, Wfc_ref[:], preferred_element_type=f32)
           + jnp.reshape(bfc_ref[:], (1, N)))
    out_ref[:] = jnp.maximum(out, 0.0)


def _full(shape):
    nd = len(shape)
    return pl.BlockSpec(shape, lambda i: (0,) * nd)


def kernel(adj_matrix, W1, as1, ad1, b1, W2, as2, ad2, b2,
           W3, as3, ad3, b3, Wfc, bfc):
    KC = H * HID
    w2_specs = [pl.BlockSpec((KC // _Q, KC), lambda i, q=q: (q, 0))
                for q in range(_Q)]
    in_specs = [
        _full((N, N)), _full((N, KC)), _full((H, HID)), _full((H, HID)),
        _full((KC,)),
        *w2_specs,
        _full((H, HID)), _full((H, HID)), _full((KC,)),
        _full((KC, HID)), _full((1, HID)), _full((1, HID)), _full((HID,)),
        _full((HID, N)), _full((N,)),
    ]
    return pl.pallas_call(
        _gat_kernel,
        out_shape=jax.ShapeDtypeStruct((N, N), jnp.float32),
        grid=(1,),
        in_specs=in_specs,
        out_specs=_full((N, N)),
    )(adj_matrix, W1, as1, ad1, b1, *([W2] * _Q), as2, ad2, b2,
      W3, as3, ad3, b3, Wfc, bfc)
